# Initial kernel scaffold; baseline (speedup 1.0000x reference)
import functools

import jax
import jax.numpy as jnp
from jax import lax
from jax.experimental import pallas as pl
from jax.experimental.pallas import tpu as pltpu
from jax.experimental.pallas import tpu_sc as plsc

EMBED_DIM = 32
BATCH = 16384
FIELDS = 26
TOTAL = BATCH * FIELDS
NUM_CORES = 2
NUM_SUBCORES = 16
NW = NUM_CORES * NUM_SUBCORES
ROWS_PER_W = TOTAL // NW
GRP = 128
G_PER_CHUNK = 8
CHUNK = GRP * G_PER_CHUNK
GROUPS_PER_W = ROWS_PER_W // GRP
N_CHUNKS = GROUPS_PER_W // G_PER_CHUNK

_mesh = plsc.VectorSubcoreMesh(core_axis_name="c", subcore_axis_name="s")


@functools.partial(
    pl.kernel,
    mesh=_mesh,
    out_type=jax.ShapeDtypeStruct((TOTAL, EMBED_DIM), jnp.float32),
    scratch_types=[
        pltpu.VMEM((G_PER_CHUNK, GRP), jnp.int32),
        pltpu.VMEM((CHUNK, EMBED_DIM), jnp.float32),
        pltpu.SemaphoreType.DMA,
    ],
)
def _sc_gather(idx_hbm, table_hbm, out_hbm, idx_v, rows_v, sem):
    wid = lax.axis_index("s") * NUM_CORES + lax.axis_index("c")

    def chunk_body(c, carry):
        grp_base = wid * GROUPS_PER_W + c * G_PER_CHUNK
        pltpu.sync_copy(idx_hbm.at[pl.ds(grp_base, G_PER_CHUNK)], idx_v)
        descs = [
            pltpu.async_copy(
                table_hbm.at[idx_v.at[g]],
                rows_v.at[pl.ds(g * GRP, GRP)],
                sem,
            )
            for g in range(G_PER_CHUNK)
        ]
        for d in descs:
            d.wait()
        pltpu.sync_copy(rows_v, out_hbm.at[pl.ds(grp_base * GRP, CHUNK)])
        return carry

    lax.fori_loop(0, N_CHUNKS, chunk_body, 0)


def kernel(indices, table):
    flat_idx = indices.astype(jnp.int32).reshape(TOTAL // GRP, GRP)
    out = _sc_gather(flat_idx, table)
    return out.reshape(BATCH, FIELDS, EMBED_DIM)


# speedup vs baseline: 1.5477x; 1.5477x over previous
import functools

import jax
import jax.numpy as jnp
from jax import lax
from jax.experimental import pallas as pl
from jax.experimental.pallas import tpu as pltpu
from jax.experimental.pallas import tpu_sc as plsc

EMBED_DIM = 32
BATCH = 16384
FIELDS = 26
TOTAL = BATCH * FIELDS
NUM_CORES = 2
NUM_SUBCORES = 16
NW = NUM_CORES * NUM_SUBCORES
ROWS_PER_W = TOTAL // NW
GRP = 128
G_PER_CHUNK = 8
CHUNK = GRP * G_PER_CHUNK
GROUPS_PER_W = ROWS_PER_W // GRP
N_CHUNKS = GROUPS_PER_W // G_PER_CHUNK

_mesh = plsc.VectorSubcoreMesh(core_axis_name="c", subcore_axis_name="s")


@functools.partial(
    pl.kernel,
    mesh=_mesh,
    out_type=jax.ShapeDtypeStruct((TOTAL, EMBED_DIM), jnp.float32),
    scratch_types=[
        pltpu.VMEM((G_PER_CHUNK, GRP), jnp.int32),
        pltpu.VMEM((CHUNK, EMBED_DIM), jnp.float32),
        pltpu.SemaphoreType.DMA,
    ],
    compiler_params=pltpu.CompilerParams(use_tc_tiling_on_sc=False),
)
def _sc_gather(idx_hbm, table_hbm, out_hbm, idx_v, rows_v, sem):
    wid = lax.axis_index("s") * NUM_CORES + lax.axis_index("c")

    def chunk_body(c, carry):
        grp_base = wid * GROUPS_PER_W + c * G_PER_CHUNK
        pltpu.sync_copy(idx_hbm.at[pl.ds(grp_base, G_PER_CHUNK)], idx_v)
        descs = [
            pltpu.async_copy(
                table_hbm.at[idx_v.at[g]],
                rows_v.at[pl.ds(g * GRP, GRP)],
                sem,
            )
            for g in range(G_PER_CHUNK)
        ]
        for d in descs:
            d.wait()
        pltpu.sync_copy(rows_v, out_hbm.at[pl.ds(grp_base * GRP, CHUNK)])
        return carry

    lax.fori_loop(0, N_CHUNKS, chunk_body, 0)


def kernel(indices, table):
    flat_idx = indices.astype(jnp.int32).reshape(TOTAL // GRP, GRP)
    out = _sc_gather(flat_idx, table)
    return out.reshape(BATCH, FIELDS, EMBED_DIM)


# speedup vs baseline: 1.5751x; 1.0177x over previous
import functools

import jax
import jax.numpy as jnp
from jax import lax
from jax.experimental import pallas as pl
from jax.experimental.pallas import tpu as pltpu
from jax.experimental.pallas import tpu_sc as plsc

EMBED_DIM = 32
BATCH = 16384
FIELDS = 26
TOTAL = BATCH * FIELDS
NUM_CORES = 2
NUM_SUBCORES = 16
NW = NUM_CORES * NUM_SUBCORES
ROWS_PER_W = TOTAL // NW
GRP = 128
G_PER_CHUNK = 13
CHUNK = GRP * G_PER_CHUNK
GROUPS_PER_W = ROWS_PER_W // GRP
N_CHUNKS = GROUPS_PER_W // G_PER_CHUNK

_mesh = plsc.VectorSubcoreMesh(core_axis_name="c", subcore_axis_name="s")


@functools.partial(
    pl.kernel,
    mesh=_mesh,
    out_type=jax.ShapeDtypeStruct((TOTAL, EMBED_DIM), jnp.float32),
    scratch_types=[
        pltpu.VMEM((GROUPS_PER_W, GRP), jnp.int32),
        pltpu.VMEM((CHUNK, EMBED_DIM), jnp.float32),
        pltpu.VMEM((CHUNK, EMBED_DIM), jnp.float32),
        pltpu.SemaphoreType.DMA,
        pltpu.SemaphoreType.DMA,
    ],
    compiler_params=pltpu.CompilerParams(use_tc_tiling_on_sc=False),
)
def _sc_gather(idx_hbm, table_hbm, out_hbm, idx_v, rows0, rows1, sem0, sem1):
    wid = lax.axis_index("s") * NUM_CORES + lax.axis_index("c")
    grp0 = wid * GROUPS_PER_W
    row0 = grp0 * GRP

    pltpu.sync_copy(idx_hbm.at[pl.ds(grp0, GROUPS_PER_W)], idx_v)

    def fire(c, buf, sem):
        for g in range(G_PER_CHUNK):
            pltpu.async_copy(
                table_hbm.at[idx_v.at[c * G_PER_CHUNK + g]],
                buf.at[pl.ds(g * GRP, GRP)],
                sem,
            )

    def drain(buf, sem):
        pltpu.make_async_copy(out_hbm.at[pl.ds(0, CHUNK)], buf, sem).wait()

    def flush(c, buf):
        pltpu.sync_copy(buf, out_hbm.at[pl.ds(row0 + c * CHUNK, CHUNK)])

    fire(0, rows0, sem0)
    fire(1, rows1, sem1)

    def step(i, carry):
        c0 = 2 * i
        drain(rows0, sem0)
        flush(c0, rows0)

        @pl.when(i < N_CHUNKS // 2 - 1)
        def _():
            fire(c0 + 2, rows0, sem0)

        drain(rows1, sem1)
        flush(c0 + 1, rows1)

        @pl.when(i < N_CHUNKS // 2 - 1)
        def _():
            fire(c0 + 3, rows1, sem1)

        return carry

    lax.fori_loop(0, N_CHUNKS // 2, step, 0)


def kernel(indices, table):
    flat_idx = indices.astype(jnp.int32).reshape(TOTAL // GRP, GRP)
    out = _sc_gather(flat_idx, table)
    return out.reshape(BATCH, FIELDS, EMBED_DIM)
